# Initial kernel scaffold; baseline (speedup 1.0000x reference)
import jax
import jax.numpy as jnp
from jax.experimental import pallas as pl
from jax.experimental.pallas import tpu as pltpu

_K = 8192
_D = 256
_TM = 256


def _vq_argmin_kernel(flat_ref, w_ref, out_ref):
    flat = flat_ref[...]
    w = w_ref[...]
    z2 = jnp.sum(flat * flat, axis=1, keepdims=True)
    w2 = jnp.sum(w * w, axis=1)
    m = jax.lax.dot_general(
        flat, w, (((1,), (1,)), ((), ())),
        preferred_element_type=jnp.float32)
    dist = (z2 - 2.0 * m) + w2[None, :]
    out_ref[...] = jnp.argmin(dist, axis=1).astype(jnp.int32)


def kernel(z_e_x, W):
    B, D, H, Wd = z_e_x.shape
    flat = jnp.transpose(z_e_x, (0, 2, 3, 1)).reshape(-1, D)
    n = flat.shape[0]
    grid = n // _TM
    out = pl.pallas_call(
        _vq_argmin_kernel,
        grid=(grid,),
        in_specs=[
            pl.BlockSpec((_TM, _D), lambda i: (i, 0)),
            pl.BlockSpec((_K, _D), lambda i: (0, 0)),
        ],
        out_specs=pl.BlockSpec((_TM,), lambda i: (i,)),
        out_shape=jax.ShapeDtypeStruct((n,), jnp.int32),
        compiler_params=pltpu.CompilerParams(
            dimension_semantics=("arbitrary",),
        ),
    )(flat, W)
    return out.reshape(B, H, Wd)


# speedup vs baseline: 1.0634x; 1.0634x over previous
import jax
import jax.numpy as jnp
from jax.experimental import pallas as pl
from jax.experimental.pallas import tpu as pltpu

_K = 8192
_D = 256
_TM = 256
_CHUNK = 4096
_NCHUNK = _K // _CHUNK


def _vq_argmin_kernel(flat_ref, w_ref, z2_ref, w2_ref, out_ref):
    flat = flat_ref[...]
    w = w_ref[...]
    z2 = z2_ref[...]
    w2 = w2_ref[...]
    m = jax.lax.dot_general(
        flat, w, (((1,), (1,)), ((), ())),
        preferred_element_type=jnp.float32)
    dist = (z2 - 2.0 * m) + w2
    acc_v = jnp.full((_TM,), jnp.inf, dtype=jnp.float32)
    acc_i = jnp.zeros((_TM,), dtype=jnp.int32)
    iota = jax.lax.broadcasted_iota(jnp.int32, (_TM, _CHUNK), 1)
    for c in range(_NCHUNK):
        sub = dist[:, c * _CHUNK:(c + 1) * _CHUNK]
        mc = jnp.min(sub, axis=1)
        ac = jnp.min(jnp.where(sub == mc[:, None], iota, _K),
                     axis=1).astype(jnp.int32) + c * _CHUNK
        take = mc < acc_v
        acc_i = jnp.where(take, ac, acc_i)
        acc_v = jnp.where(take, mc, acc_v)
        acc_v = acc_v.astype(jnp.bfloat16).astype(jnp.float32)
    out_ref[...] = acc_i


def kernel(z_e_x, W):
    B, D, H, Wd = z_e_x.shape
    flat = jnp.transpose(z_e_x, (0, 2, 3, 1)).reshape(-1, D)
    n = flat.shape[0]
    z2 = jnp.sum(flat * flat, axis=1, keepdims=True)
    w2 = jnp.sum(W * W, axis=1)[None, :]
    grid = n // _TM
    out = pl.pallas_call(
        _vq_argmin_kernel,
        grid=(grid,),
        in_specs=[
            pl.BlockSpec((_TM, _D), lambda i: (i, 0)),
            pl.BlockSpec((_K, _D), lambda i: (0, 0)),
            pl.BlockSpec((_TM, 1), lambda i: (i, 0)),
            pl.BlockSpec((1, _K), lambda i: (0, 0)),
        ],
        out_specs=pl.BlockSpec((_TM,), lambda i: (i,)),
        out_shape=jax.ShapeDtypeStruct((n,), jnp.int32),
        compiler_params=pltpu.CompilerParams(
            dimension_semantics=("arbitrary",),
        ),
    )(flat, W, z2, w2)
    return out.reshape(B, H, Wd)


# speedup vs baseline: 1.0644x; 1.0010x over previous
import jax
import jax.numpy as jnp
from jax.experimental import pallas as pl
from jax.experimental.pallas import tpu as pltpu

_K = 8192
_D = 256
_TM = 256
_CHUNK = 4096
_NCHUNK = _K // _CHUNK


def _vq_argmin_kernel(flat_ref, w_ref, z2_ref, w2_ref, out_ref):
    flat = flat_ref[...]
    w = w_ref[...]
    z2 = z2_ref[...]
    w2 = w2_ref[...]
    m = jax.lax.dot_general(
        flat, w, (((1,), (1,)), ((), ())),
        preferred_element_type=jnp.float32)
    dist = (z2 - 2.0 * m) + w2
    acc_v = jnp.full((_TM,), jnp.inf, dtype=jnp.float32)
    acc_i = jnp.zeros((_TM,), dtype=jnp.int32)
    iota = jax.lax.broadcasted_iota(jnp.int32, (_TM, _CHUNK), 1)
    for c in range(_NCHUNK):
        sub = dist[:, c * _CHUNK:(c + 1) * _CHUNK]
        mc = jnp.min(sub, axis=1)
        ac = jnp.min(jnp.where(sub == mc[:, None], iota, _K),
                     axis=1).astype(jnp.int32) + c * _CHUNK
        take = mc < acc_v
        acc_i = jnp.where(take, ac, acc_i)
        acc_v = jnp.where(take, mc, acc_v)
        acc_v = acc_v.astype(jnp.bfloat16).astype(jnp.float32)
    out_ref[...] = acc_i


def kernel(z_e_x, W):
    B, D, H, Wd = z_e_x.shape
    flat = jnp.transpose(z_e_x, (0, 2, 3, 1)).reshape(-1, D)
    n = flat.shape[0]
    z2 = jnp.sum(flat * flat, axis=1, keepdims=True)
    w2 = jnp.sum(W * W, axis=1)[None, :]
    grid = n // _TM
    out = pl.pallas_call(
        _vq_argmin_kernel,
        grid=(grid,),
        in_specs=[
            pl.BlockSpec((_TM, _D), lambda i: (i, 0)),
            pl.BlockSpec((_K, _D), lambda i: (0, 0)),
            pl.BlockSpec((_TM, 1), lambda i: (i, 0)),
            pl.BlockSpec((1, _K), lambda i: (0, 0)),
        ],
        out_specs=pl.BlockSpec((_TM,), lambda i: (i,)),
        out_shape=jax.ShapeDtypeStruct((n,), jnp.int32),
        compiler_params=pltpu.CompilerParams(
            dimension_semantics=("parallel",),
        ),
    )(flat, W, z2, w2)
    return out.reshape(B, H, Wd)


# speedup vs baseline: 1.1322x; 1.0636x over previous
import jax
import jax.numpy as jnp
from jax.experimental import pallas as pl
from jax.experimental.pallas import tpu as pltpu

_K = 8192
_D = 256
_TM = 512
_CHUNK = 4096
_NCHUNK = _K // _CHUNK


def _vq_argmin_kernel(flat_ref, w_ref, z2_ref, w2_ref, out_ref):
    flat2 = flat_ref[...] * 2.0
    z2 = z2_ref[...]
    acc_v = jnp.full((_TM,), jnp.inf, dtype=jnp.float32)
    acc_i = jnp.zeros((_TM,), dtype=jnp.int32)
    iota = jax.lax.broadcasted_iota(jnp.int32, (_TM, _CHUNK), 1)
    for c in range(_NCHUNK):
        wc = w_ref[c * _CHUNK:(c + 1) * _CHUNK, :]
        m2 = jax.lax.dot_general(
            flat2, wc, (((1,), (1,)), ((), ())),
            preferred_element_type=jnp.float32)
        sub = (z2 - m2) + w2_ref[:, c * _CHUNK:(c + 1) * _CHUNK]
        mc = jnp.min(sub, axis=1)
        ac = jnp.min(jnp.where(sub == mc[:, None], iota, _K),
                     axis=1).astype(jnp.int32) + c * _CHUNK
        take = mc < acc_v
        acc_i = jnp.where(take, ac, acc_i)
        acc_v = jnp.where(take, mc, acc_v)
        acc_v = acc_v.astype(jnp.bfloat16).astype(jnp.float32)
    out_ref[...] = acc_i


def kernel(z_e_x, W):
    B, D, H, Wd = z_e_x.shape
    flat = jnp.transpose(z_e_x, (0, 2, 3, 1)).reshape(-1, D)
    n = flat.shape[0]
    z2 = jnp.sum(flat * flat, axis=1, keepdims=True)
    w2 = jnp.sum(W * W, axis=1)[None, :]
    grid = n // _TM
    out = pl.pallas_call(
        _vq_argmin_kernel,
        grid=(grid,),
        in_specs=[
            pl.BlockSpec((_TM, _D), lambda i: (i, 0)),
            pl.BlockSpec((_K, _D), lambda i: (0, 0)),
            pl.BlockSpec((_TM, 1), lambda i: (i, 0)),
            pl.BlockSpec((1, _K), lambda i: (0, 0)),
        ],
        out_specs=pl.BlockSpec((_TM,), lambda i: (i,)),
        out_shape=jax.ShapeDtypeStruct((n,), jnp.int32),
        compiler_params=pltpu.CompilerParams(
            dimension_semantics=("parallel",),
        ),
    )(flat, W, z2, w2)
    return out.reshape(B, H, Wd)
